# Initial kernel scaffold; baseline (speedup 1.0000x reference)
import functools

import jax
import jax.numpy as jnp
from jax import lax
from jax.experimental import pallas as pl
from jax.experimental.pallas import tpu as pltpu
from jax.experimental.pallas import tpu_sc as plsc

N = 10000
E = 320000
D = 128

NC = 2
NS = 16
NW = NC * NS
EPW = E // NW
K = 128
NCHUNK = EPW // K
TAIL = EPW - NCHUNK * K
RPT = N // NS
CW = 16
NPAD = NS * 640


def _sc_body(x_hbm, src_hbm, dst_hbm, acc_out, cnt_out,
             src_v, dst_v, rows_v, ones_v, srct_v, dstt_v, rowst_v,
             acc_sh, cnt_sh, sem):
    c = lax.axis_index("c")
    s = lax.axis_index("s")
    w = s * NC + c
    ebase = w * EPW

    zeros16 = jnp.zeros((16,), jnp.float32)
    ones16 = jnp.ones((16,), jnp.float32)

    def _fill(r, carry):
        ones_v[r, :] = ones16
        for j in range(D // 16):
            rows_v[r, pl.ds(j * 16, 16)] = zeros16
        return carry

    lax.fori_loop(0, K, _fill, 0)

    rbase = s * RPT
    for t in range(RPT // K):
        pltpu.sync_copy(rows_v, acc_sh.at[pl.ds(rbase + t * K, K), :])
    rem = RPT - (RPT // K) * K
    pltpu.sync_copy(rows_v.at[pl.ds(0, rem)],
                    acc_sh.at[pl.ds(rbase + (RPT // K) * K, rem), :])
    for t in range(5):
        pltpu.sync_copy(rows_v.at[:, pl.ds(0, CW)],
                        cnt_sh.at[pl.ds(s * 640 + t * K, K), :])
    plsc.subcore_barrier()

    def _chunk(i, carry):
        base = pl.multiple_of(ebase + i * K, 8)
        pltpu.sync_copy(src_hbm.at[pl.ds(base, K)], src_v)
        pltpu.sync_copy(dst_hbm.at[pl.ds(base, K)], dst_v)
        pltpu.async_copy(x_hbm.at[src_v], rows_v, sem).wait()
        pltpu.sync_copy(rows_v, acc_sh.at[dst_v], add=True)
        pltpu.sync_copy(ones_v, cnt_sh.at[dst_v], add=True)
        return carry

    lax.fori_loop(0, NCHUNK, _chunk, 0)

    tbase = ebase + NCHUNK * K
    pltpu.sync_copy(src_hbm.at[pl.ds(tbase, TAIL)], srct_v)
    pltpu.sync_copy(dst_hbm.at[pl.ds(tbase, TAIL)], dstt_v)
    pltpu.async_copy(x_hbm.at[srct_v], rowst_v, sem).wait()
    pltpu.sync_copy(rowst_v, acc_sh.at[dstt_v], add=True)
    pltpu.sync_copy(ones_v.at[pl.ds(0, TAIL)], cnt_sh.at[dstt_v], add=True)

    plsc.subcore_barrier()

    pltpu.sync_copy(acc_sh.at[pl.ds(rbase, RPT)],
                    acc_out.at[c, pl.ds(rbase, RPT), :])
    pltpu.sync_copy(cnt_sh.at[pl.ds(s * 640, 640)],
                    cnt_out.at[c, pl.ds(s * 640, 640), :])


_sc_scatter = functools.partial(
    pl.kernel,
    mesh=plsc.VectorSubcoreMesh(core_axis_name="c", subcore_axis_name="s"),
    out_type=[
        jax.ShapeDtypeStruct((NC, N, D), jnp.float32),
        jax.ShapeDtypeStruct((NC, NPAD, CW), jnp.float32),
    ],
    scratch_types=[
        pltpu.VMEM((K,), jnp.int32),
        pltpu.VMEM((K,), jnp.int32),
        pltpu.VMEM((K, D), jnp.float32),
        pltpu.VMEM((K, CW), jnp.float32),
        pltpu.VMEM((TAIL,), jnp.int32),
        pltpu.VMEM((TAIL,), jnp.int32),
        pltpu.VMEM((TAIL, D), jnp.float32),
        pltpu.VMEM_SHARED((N, D), jnp.float32),
        pltpu.VMEM_SHARED((NPAD, CW), jnp.float32),
        pltpu.SemaphoreType.DMA,
    ],
)(_sc_body)


BLK = 400


def _tc_body(x_ref, p_ref, c_ref, wl_ref, wr_ref, b_ref, o_ref):
    x = x_ref[...]
    p = p_ref[0] + p_ref[1]
    cnt = c_ref[0][:, 0:1] + c_ref[1][:, 0:1]
    mean = p / jnp.maximum(cnt, 1.0)
    y = (jnp.dot(mean, wl_ref[...], preferred_element_type=jnp.float32)
         + jnp.dot(x, wr_ref[...], preferred_element_type=jnp.float32)
         + b_ref[...])
    o_ref[...] = x + y


def _tc_finish(x, acc, cnt, wl_t, wr_t, b_row):
    return pl.pallas_call(
        _tc_body,
        grid=(N // BLK,),
        in_specs=[
            pl.BlockSpec((BLK, D), lambda i: (i, 0)),
            pl.BlockSpec((NC, BLK, D), lambda i: (0, i, 0)),
            pl.BlockSpec((NC, BLK, CW), lambda i: (0, i, 0)),
            pl.BlockSpec((D, D), lambda i: (0, 0)),
            pl.BlockSpec((D, D), lambda i: (0, 0)),
            pl.BlockSpec((1, D), lambda i: (0, 0)),
        ],
        out_specs=pl.BlockSpec((BLK, D), lambda i: (i, 0)),
        out_shape=jax.ShapeDtypeStruct((N, D), jnp.float32),
    )(x, acc, cnt, wl_t, wr_t, b_row)


def kernel(x, edge_index, W_l, b_l, W_r):
    x = x.astype(jnp.float32)
    ei = edge_index.astype(jnp.int32)
    acc, cnt = _sc_scatter(x, ei[0], ei[1])
    return _tc_finish(x, acc, cnt, W_l.T, W_r.T, b_l.reshape(1, D))


# speedup vs baseline: 7.7582x; 7.7582x over previous
import functools

import jax
import jax.numpy as jnp
from jax import lax
from jax.experimental import pallas as pl
from jax.experimental.pallas import tpu as pltpu
from jax.experimental.pallas import tpu_sc as plsc

N = 10000
E = 320000
D = 128

NC = 2
NS = 16
NW = NC * NS
EPW = E // NW
K = 128
NCHUNK = EPW // K
TAIL = EPW - NCHUNK * K
CW = 16
NPAD = NS * 640
RPT = NPAD // NS


def _sc_body(x_hbm, src_hbm, dst_hbm, acc_out, cnt_out,
             src_v, dst_v, rows_v, ones_v, zcnt_v, srct_v, dstt_v, rowst_v,
             acc_sh, cnt_sh, sem):
    c = lax.axis_index("c")
    s = lax.axis_index("s")
    w = s * NC + c
    ebase = w * EPW

    zeros16 = jnp.zeros((16,), jnp.float32)
    ones16 = jnp.ones((16,), jnp.float32)

    def _fill(r, carry):
        for j in range(D // 16):
            rows_v[r, pl.ds(j * 16, 16)] = zeros16
        return carry

    lax.fori_loop(0, K, _fill, 0)

    def _fill1(r, carry):
        ones_v[pl.ds(r * 16, 16)] = ones16
        return carry

    lax.fori_loop(0, K // 16, _fill1, 0)

    def _fillz(r, carry):
        zcnt_v[pl.ds(r * 16, 16)] = zeros16
        return carry

    lax.fori_loop(0, RPT // 16, _fillz, 0)

    rbase = pl.multiple_of(s * RPT, 8)
    for t in range(RPT // K):
        pltpu.sync_copy(rows_v, acc_sh.at[pl.ds(rbase + t * K, K), :])
    pltpu.sync_copy(zcnt_v, cnt_sh.at[pl.ds(rbase, RPT)])
    plsc.subcore_barrier()

    def _chunk(i, carry):
        base = pl.multiple_of(ebase + i * K, 8)
        pltpu.sync_copy(src_hbm.at[pl.ds(base, K)], src_v)
        pltpu.sync_copy(dst_hbm.at[pl.ds(base, K)], dst_v)
        pltpu.async_copy(x_hbm.at[src_v], rows_v, sem).wait()
        pltpu.sync_copy(rows_v, acc_sh.at[dst_v], add=True)
        pltpu.sync_copy(ones_v, cnt_sh.at[dst_v], add=True)
        return carry

    lax.fori_loop(0, NCHUNK, _chunk, 0)

    tbase = ebase + NCHUNK * K
    pltpu.sync_copy(src_hbm.at[pl.ds(tbase, TAIL)], srct_v)
    pltpu.sync_copy(dst_hbm.at[pl.ds(tbase, TAIL)], dstt_v)
    pltpu.async_copy(x_hbm.at[srct_v], rowst_v, sem).wait()
    pltpu.sync_copy(rowst_v, acc_sh.at[dstt_v], add=True)
    pltpu.sync_copy(ones_v.at[pl.ds(0, TAIL)], cnt_sh.at[dstt_v], add=True)

    plsc.subcore_barrier()

    pltpu.sync_copy(acc_sh.at[pl.ds(rbase, RPT)],
                    acc_out.at[c, pl.ds(rbase, RPT), :])
    pltpu.sync_copy(cnt_sh.at[pl.ds(rbase, RPT)],
                    cnt_out.at[c, pl.ds(rbase, RPT)])


_sc_scatter = functools.partial(
    pl.kernel,
    mesh=plsc.VectorSubcoreMesh(core_axis_name="c", subcore_axis_name="s"),
    out_type=[
        jax.ShapeDtypeStruct((NC, NPAD, D), jnp.float32),
        jax.ShapeDtypeStruct((NC, NPAD), jnp.float32),
    ],
    scratch_types=[
        pltpu.VMEM((K,), jnp.int32),
        pltpu.VMEM((K,), jnp.int32),
        pltpu.VMEM((K, D), jnp.float32),
        pltpu.VMEM((K,), jnp.float32),
        pltpu.VMEM((RPT,), jnp.float32),
        pltpu.VMEM((TAIL,), jnp.int32),
        pltpu.VMEM((TAIL,), jnp.int32),
        pltpu.VMEM((TAIL, D), jnp.float32),
        pltpu.VMEM_SHARED((NPAD, D), jnp.float32),
        pltpu.VMEM_SHARED((NPAD,), jnp.float32),
        pltpu.SemaphoreType.DMA,
    ],
)(_sc_body)


BLK = 512


def _tc_body(x_ref, p_ref, c_ref, wl_ref, wr_ref, b_ref, o_ref):
    x = x_ref[...]
    p = p_ref[0] + p_ref[1]
    cnt = c_ref[0] + c_ref[1]
    mean = p / jnp.maximum(cnt, 1.0)[:, None]
    y = (jnp.dot(mean, wl_ref[...], preferred_element_type=jnp.float32)
         + jnp.dot(x, wr_ref[...], preferred_element_type=jnp.float32)
         + b_ref[...])
    o_ref[...] = x + y


def _tc_finish(x, acc, cnt, wl_t, wr_t, b_row):
    return pl.pallas_call(
        _tc_body,
        grid=(pl.cdiv(N, BLK),),
        in_specs=[
            pl.BlockSpec((BLK, D), lambda i: (i, 0)),
            pl.BlockSpec((NC, BLK, D), lambda i: (0, i, 0)),
            pl.BlockSpec((NC, BLK), lambda i: (0, i)),
            pl.BlockSpec((D, D), lambda i: (0, 0)),
            pl.BlockSpec((D, D), lambda i: (0, 0)),
            pl.BlockSpec((1, D), lambda i: (0, 0)),
        ],
        out_specs=pl.BlockSpec((BLK, D), lambda i: (i, 0)),
        out_shape=jax.ShapeDtypeStruct((N, D), jnp.float32),
    )(x, acc, cnt, wl_t, wr_t, b_row)


def kernel(x, edge_index, W_l, b_l, W_r):
    x = x.astype(jnp.float32)
    ei = edge_index.astype(jnp.int32)
    acc, cnt = _sc_scatter(x, ei[0], ei[1])
    return _tc_finish(x, acc, cnt, W_l.T, W_r.T, b_l.reshape(1, D))
